# Initial kernel scaffold; baseline (speedup 1.0000x reference)
import jax
import jax.numpy as jnp
from jax.experimental import pallas as pl


def kernel(x, edge_index, W_in, b_in, g_in, be_in, W_l, a_src, a_dst, b_l, g_l, be_l, Wc1, bc1, g1, be1, Wc2, bc2, g2, be2, Wc3, bc3):
    raise NotImplementedError("write your pallas kernel here")


# speedup vs baseline: 19.4483x; 19.4483x over previous
import functools

import jax
import jax.numpy as jnp
from jax import lax
from jax.experimental import pallas as pl
from jax.experimental.pallas import tpu as pltpu
from jax.experimental.pallas import tpu_sc as plsc

N = 10000
E = 320000
H = 128
HEADS = 8
DH = 16

NB = 10
BR = N // NB
NCORE = 2
NSUB = 16
RPT = N // NSUB
ECORE = E // NCORE
SCAN = 256
NSCAN = ECORE // SCAN
GK = 96
DEN_W = RPT * 8 + 8


def _ln_relu(z, g, b):
    m = jnp.mean(z, axis=1, keepdims=True)
    v = jnp.mean((z - m) ** 2, axis=1, keepdims=True)
    return jnp.maximum((z - m) / jnp.sqrt(v + 1e-5) * g + b, 0.0)


def _tc_in_body(x_ref, w_ref, b_ref, g_ref, be_ref, o_ref):
    h = jnp.dot(x_ref[...], w_ref[...], preferred_element_type=jnp.float32)
    o_ref[...] = _ln_relu(h + b_ref[...], g_ref[...], be_ref[...])


def _tc_in(x, W, b, g, be):
    full = lambda r, c: pl.BlockSpec((r, c), lambda i: (0, 0))
    return pl.pallas_call(
        _tc_in_body,
        grid=(NB,),
        in_specs=[pl.BlockSpec((BR, H), lambda i: (i, 0)),
                  full(H, H), full(1, H), full(1, H), full(1, H)],
        out_specs=pl.BlockSpec((BR, H), lambda i: (i, 0)),
        out_shape=jax.ShapeDtypeStruct((N, H), jnp.float32),
    )(x, W, b.reshape(1, H), g.reshape(1, H), be.reshape(1, H))


def _tc_pre_body(h_ref, w_ref, as_ref, ad_ref, r_ref,
                 hw_ref, s_ref, ws_ref, accs_ref, dp_ref):
    hw = jnp.dot(h_ref[...], w_ref[...], preferred_element_type=jnp.float32)
    s = jnp.dot(hw, as_ref[...], preferred_element_type=jnp.float32)
    d = jnp.dot(hw, ad_ref[...], preferred_element_type=jnp.float32)
    e = s + d
    w = jnp.exp(jnp.maximum(e, 0.2 * e))
    hw_ref[...] = hw
    s_ref[...] = s
    ws_ref[...] = w
    accs_ref[...] = hw * jnp.dot(w, r_ref[...], preferred_element_type=jnp.float32)
    dp_ref[...] = d


def _tc_pre(h, W, As, Ad, R):
    full = lambda r, c: pl.BlockSpec((r, c), lambda i: (0, 0))
    blk = lambda c: pl.BlockSpec((BR, c), lambda i: (i, 0))
    return pl.pallas_call(
        _tc_pre_body,
        grid=(NB,),
        in_specs=[blk(H), full(H, H), full(H, HEADS), full(H, HEADS),
                  full(HEADS, H)],
        out_specs=[blk(H), blk(HEADS), blk(HEADS), blk(H), blk(HEADS)],
        out_shape=[jax.ShapeDtypeStruct((N, H), jnp.float32),
                   jax.ShapeDtypeStruct((N, HEADS), jnp.float32),
                   jax.ShapeDtypeStruct((N, HEADS), jnp.float32),
                   jax.ShapeDtypeStruct((N, H), jnp.float32),
                   jax.ShapeDtypeStruct((N, HEADS), jnp.float32)],
    )(h, W, As, Ad, R)


def _tc_post_body(a0_ref, a1_ref, d0_ref, d1_ref, hres_ref,
                  b_ref, g_ref, be_ref, r_ref, o_ref, *, add_resid):
    den = d0_ref[...] + d1_ref[...] + 1e-16
    acc = a0_ref[...] + a1_ref[...]
    dexp = jnp.dot(den, r_ref[...], preferred_element_type=jnp.float32)
    z = _ln_relu(acc / dexp + b_ref[...], g_ref[...], be_ref[...])
    if add_resid:
        z = z + hres_ref[...]
    o_ref[...] = z


def _tc_post(a0, a1, d0, d1, hres, b, g, be, R, add_resid):
    full = lambda r, c: pl.BlockSpec((r, c), lambda i: (0, 0))
    blk = lambda c: pl.BlockSpec((BR, c), lambda i: (i, 0))
    return pl.pallas_call(
        functools.partial(_tc_post_body, add_resid=add_resid),
        grid=(NB,),
        in_specs=[blk(H), blk(H), blk(HEADS), blk(HEADS),
                  blk(H), full(1, H), full(1, H), full(1, H), full(HEADS, H)],
        out_specs=blk(H),
        out_shape=jax.ShapeDtypeStruct((N, H), jnp.float32),
    )(a0, a1, d0, d1, hres,
      b.reshape(1, H), g.reshape(1, H), be.reshape(1, H), R)


def _tc_cls_body(h_ref, w1_ref, b1_ref, g1_ref, be1_ref,
                 w2_ref, b2_ref, g2_ref, be2_ref, w3_ref, b3_ref, o_ref):
    y = jnp.dot(h_ref[...], w1_ref[...], preferred_element_type=jnp.float32)
    y = _ln_relu(y + b1_ref[...], g1_ref[...], be1_ref[...])
    y = jnp.dot(y, w2_ref[...], preferred_element_type=jnp.float32)
    y = _ln_relu(y + b2_ref[...], g2_ref[...], be2_ref[...])
    y = jnp.dot(y, w3_ref[...], preferred_element_type=jnp.float32)
    o_ref[...] = y + b3_ref[...]


def _tc_cls(h, Wc1, bc1, g1, be1, Wc2, bc2, g2, be2, Wc3, bc3):
    full = lambda r, c: pl.BlockSpec((r, c), lambda i: (0, 0))
    blk = lambda c: pl.BlockSpec((BR, c), lambda i: (i, 0))
    h2, h4, nc = H // 2, H // 4, 8
    return pl.pallas_call(
        _tc_cls_body,
        grid=(NB,),
        in_specs=[blk(H), full(H, h2), full(1, h2), full(1, h2), full(1, h2),
                  full(h2, h4), full(1, h4), full(1, h4), full(1, h4),
                  full(h4, nc), full(1, nc)],
        out_specs=blk(nc),
        out_shape=jax.ShapeDtypeStruct((N, nc), jnp.float32),
    )(h, Wc1, bc1.reshape(1, h2), g1.reshape(1, h2), be1.reshape(1, h2),
      Wc2, bc2.reshape(1, h4), g2.reshape(1, h4), be2.reshape(1, h4),
      Wc3, bc3.reshape(1, nc))


_MESH = plsc.VectorSubcoreMesh(core_axis_name="c", subcore_axis_name="s",
                               num_cores=NCORE, num_subcores=NSUB)
_GD = lax.GatherDimensionNumbers(offset_dims=(), collapsed_slice_dims=(0,),
                                 start_index_map=(0,))


@functools.partial(
    pl.kernel,
    out_type=(jax.ShapeDtypeStruct((NCORE, NSUB, RPT * H), jnp.float32),
              jax.ShapeDtypeStruct((NCORE, NSUB, DEN_W), jnp.float32)),
    mesh=_MESH,
    scratch_types=[
        pltpu.VMEM((RPT * H,), jnp.float32),
        pltpu.VMEM((DEN_W,), jnp.float32),
        pltpu.VMEM((DEN_W,), jnp.float32),
        pltpu.VMEM((32,), jnp.int32),
        pltpu.VMEM((GK,), jnp.int32),
        pltpu.VMEM((GK + 16,), jnp.int32),
        pltpu.VMEM((GK * 8 + 16,), jnp.float32),
        pltpu.VMEM((GK, H), jnp.float32),
        pltpu.SemaphoreType.DMA,
    ],
)
def _sc_edge(hw_hbm, ps_hbm, dtab_hbm, psrc_hbm, pdl_hbm, sbuf_hbm,
             acc0_hbm, den0_hbm, acc_out, den_out,
             acc_l, den_l, d_l, sbuf_v, idx_v, dl_v, ps_l, hwrow_v, sem):
    cid = lax.axis_index("c")
    sid = lax.axis_index("s")
    lane = lax.iota(jnp.int32, 16)

    pltpu.sync_copy(acc0_hbm.at[cid, sid], acc_l)
    pltpu.sync_copy(den0_hbm.at[cid, sid], den_l)
    pltpu.sync_copy(dtab_hbm.at[sid], d_l)
    pltpu.sync_copy(sbuf_hbm, sbuf_v)

    st = sbuf_v[pl.ds(sid, 16)][0]
    en = sbuf_v[pl.ds(sid + 1, 16)][0]
    h0 = (en - st) // 2
    my_lo = st + cid * h0
    my_hi = jnp.where(cid == 0, st + h0, en)
    base8 = (my_lo // 8) * 8

    @pl.loop(0, (my_hi - base8 + GK - 1) // GK)
    def _g(g):
        cb = base8 + g * GK
        pltpu.sync_copy(psrc_hbm.at[pl.ds(cb, GK)], idx_v)
        pltpu.sync_copy(pdl_hbm.at[pl.ds(cb, GK)], dl_v.at[pl.ds(0, GK)])
        pltpu.sync_copy(ps_hbm.at[pl.ds(cb * 8, GK * 8)],
                        ps_l.at[pl.ds(0, GK * 8)])
        pltpu.async_copy(hw_hbm.at[idx_v], hwrow_v, sem).wait()
        jlo = jnp.maximum(my_lo - cb, 0)
        jhi = jnp.minimum(my_hi - cb, GK)

        @pl.loop(jlo, jhi)
        def _e(j):
            dl = dl_v[pl.ds(j, 16)][0]
            sv = ps_l[pl.ds(j * 8, 16)]
            dv = d_l[pl.ds(dl * 8, 16)]
            e = sv + dv
            w16 = jnp.exp(jnp.maximum(e, 0.2 * e))
            w16 = jnp.where(lane < 8, w16, 0.0)
            db = den_l[pl.ds(dl * 8, 16)]
            den_l[pl.ds(dl * 8, 16)] = db + w16
            for c in range(8):
                cv = jnp.full((16,), c, jnp.int32)
                b = lax.gather(w16, cv[:, None], _GD, (1,),
                               mode=lax.GatherScatterMode.PROMISE_IN_BOUNDS)
                acc_l[pl.ds(dl * H + c * 16, 16)] = (
                    acc_l[pl.ds(dl * H + c * 16, 16)]
                    + hwrow_v[j, pl.ds(c * 16, 16)] * b)

    pltpu.sync_copy(acc_l, acc_out.at[cid, sid])
    pltpu.sync_copy(den_l, den_out.at[cid, sid])


def kernel(x, edge_index, W_in, b_in, g_in, be_in, W_l, a_src, a_dst, b_l,
           g_l, be_l, Wc1, bc1, g1, be1, Wc2, bc2, g2, be2, Wc3, bc3):
    src = edge_index[0]
    dst = edge_index[1]
    key = dst // RPT
    perm = jnp.argsort(key, stable=True)
    pad = jnp.zeros((GK + 8,), jnp.int32)
    psrc = jnp.concatenate([src[perm], pad])
    pdl = jnp.concatenate([dst[perm] - key[perm] * RPT, pad])
    skey = key[perm]
    starts = jnp.searchsorted(skey, jnp.arange(NSUB + 1, dtype=jnp.int32),
                              side="left").astype(jnp.int32)
    sbuf = jnp.concatenate([starts, jnp.zeros((32 - NSUB - 1,), jnp.int32)])
    eye = jnp.eye(HEADS, dtype=jnp.float32)
    R = jnp.repeat(eye, DH, axis=1)

    h = _tc_in(x, W_in, b_in, g_in, be_in)
    for i in range(3):
        As = (eye[:, None, :] * a_src[i][:, :, None]).reshape(H, HEADS)
        Ad = (eye[:, None, :] * a_dst[i][:, :, None]).reshape(H, HEADS)
        hw, s, ws, accs, dpad = _tc_pre(h, W_l[i], As, Ad, R)
        ps = jnp.concatenate([jnp.take(s, perm, axis=0).reshape(-1),
                              jnp.zeros(((GK + 8) * 8,), jnp.float32)])
        dtab = jnp.pad(dpad.reshape(NSUB, RPT * 8), ((0, 0), (0, 8)))
        acc_init = jnp.stack([accs.reshape(NSUB, RPT * H),
                              jnp.zeros((NSUB, RPT * H), jnp.float32)])
        den_init = jnp.stack([
            jnp.pad(ws.reshape(NSUB, RPT * 8), ((0, 0), (0, 8))),
            jnp.zeros((NSUB, DEN_W), jnp.float32)])
        accp, denp = _sc_edge(hw, ps, dtab, psrc, pdl, sbuf,
                              acc_init, den_init)
        accp = accp.reshape(NCORE, N, H)
        denp = denp[:, :, :RPT * 8].reshape(NCORE, N, HEADS)
        h = _tc_post(accp[0], accp[1], denp[0], denp[1], h,
                     b_l[i], g_l[i], be_l[i], R, add_resid=(i > 0))
    return _tc_cls(h, Wc1, bc1, g1, be1, Wc2, bc2, g2, be2, Wc3, bc3)


# speedup vs baseline: 19.4655x; 1.0009x over previous
import functools

import jax
import jax.numpy as jnp
from jax import lax
from jax.experimental import pallas as pl
from jax.experimental.pallas import tpu as pltpu
from jax.experimental.pallas import tpu_sc as plsc

N = 10000
E = 320000
H = 128
HEADS = 8
DH = 16

NB = 10
BR = N // NB
NCORE = 2
NSUB = 16
RPT = N // NSUB
GK = 96
DEN_W = RPT * 8 + 8


def _ln_relu(z, g, b):
    m = jnp.mean(z, axis=1, keepdims=True)
    v = jnp.mean((z - m) ** 2, axis=1, keepdims=True)
    return jnp.maximum((z - m) / jnp.sqrt(v + 1e-5) * g + b, 0.0)


def _tc_in_body(x_ref, w_ref, b_ref, g_ref, be_ref, o_ref):
    h = jnp.dot(x_ref[...], w_ref[...], preferred_element_type=jnp.float32)
    o_ref[...] = _ln_relu(h + b_ref[...], g_ref[...], be_ref[...])


def _tc_in(x, W, b, g, be):
    full = lambda r, c: pl.BlockSpec((r, c), lambda i: (0, 0))
    return pl.pallas_call(
        _tc_in_body,
        grid=(NB,),
        in_specs=[pl.BlockSpec((BR, H), lambda i: (i, 0)),
                  full(H, H), full(1, H), full(1, H), full(1, H)],
        out_specs=pl.BlockSpec((BR, H), lambda i: (i, 0)),
        out_shape=jax.ShapeDtypeStruct((N, H), jnp.float32),
    )(x, W, b.reshape(1, H), g.reshape(1, H), be.reshape(1, H))


def _tc_pre_body(h_ref, w_ref, as_ref, ad_ref, r_ref,
                 hw_ref, s_ref, ws_ref, accs_ref, dp_ref):
    hw = jnp.dot(h_ref[...], w_ref[...], preferred_element_type=jnp.float32)
    s = jnp.dot(hw, as_ref[...], preferred_element_type=jnp.float32)
    d = jnp.dot(hw, ad_ref[...], preferred_element_type=jnp.float32)
    e = s + d
    w = jnp.exp(jnp.maximum(e, 0.2 * e))
    hw_ref[...] = hw
    s_ref[...] = s
    ws_ref[...] = w
    accs_ref[...] = hw * jnp.dot(w, r_ref[...], preferred_element_type=jnp.float32)
    dp_ref[...] = d


def _tc_pre(h, W, As, Ad, R):
    full = lambda r, c: pl.BlockSpec((r, c), lambda i: (0, 0))
    blk = lambda c: pl.BlockSpec((BR, c), lambda i: (i, 0))
    return pl.pallas_call(
        _tc_pre_body,
        grid=(NB,),
        in_specs=[blk(H), full(H, H), full(H, HEADS), full(H, HEADS),
                  full(HEADS, H)],
        out_specs=[blk(H), blk(HEADS), blk(HEADS), blk(H), blk(HEADS)],
        out_shape=[jax.ShapeDtypeStruct((N, H), jnp.float32),
                   jax.ShapeDtypeStruct((N, HEADS), jnp.float32),
                   jax.ShapeDtypeStruct((N, HEADS), jnp.float32),
                   jax.ShapeDtypeStruct((N, H), jnp.float32),
                   jax.ShapeDtypeStruct((N, HEADS), jnp.float32)],
    )(h, W, As, Ad, R)


def _tc_post_body(a0_ref, a1_ref, d0_ref, d1_ref, hres_ref,
                  b_ref, g_ref, be_ref, r_ref, o_ref, *, add_resid):
    den = d0_ref[...] + d1_ref[...] + 1e-16
    acc = a0_ref[...] + a1_ref[...]
    dexp = jnp.dot(den, r_ref[...], preferred_element_type=jnp.float32)
    z = _ln_relu(acc / dexp + b_ref[...], g_ref[...], be_ref[...])
    if add_resid:
        z = z + hres_ref[...]
    o_ref[...] = z


def _tc_post(a0, a1, d0, d1, hres, b, g, be, R, add_resid):
    full = lambda r, c: pl.BlockSpec((r, c), lambda i: (0, 0))
    blk = lambda c: pl.BlockSpec((BR, c), lambda i: (i, 0))
    return pl.pallas_call(
        functools.partial(_tc_post_body, add_resid=add_resid),
        grid=(NB,),
        in_specs=[blk(H), blk(H), blk(HEADS), blk(HEADS),
                  blk(H), full(1, H), full(1, H), full(1, H), full(HEADS, H)],
        out_specs=blk(H),
        out_shape=jax.ShapeDtypeStruct((N, H), jnp.float32),
    )(a0, a1, d0, d1, hres,
      b.reshape(1, H), g.reshape(1, H), be.reshape(1, H), R)


def _tc_cls_body(h_ref, w1_ref, b1_ref, g1_ref, be1_ref,
                 w2_ref, b2_ref, g2_ref, be2_ref, w3_ref, b3_ref, o_ref):
    y = jnp.dot(h_ref[...], w1_ref[...], preferred_element_type=jnp.float32)
    y = _ln_relu(y + b1_ref[...], g1_ref[...], be1_ref[...])
    y = jnp.dot(y, w2_ref[...], preferred_element_type=jnp.float32)
    y = _ln_relu(y + b2_ref[...], g2_ref[...], be2_ref[...])
    y = jnp.dot(y, w3_ref[...], preferred_element_type=jnp.float32)
    o_ref[...] = y + b3_ref[...]


def _tc_cls(h, Wc1, bc1, g1, be1, Wc2, bc2, g2, be2, Wc3, bc3):
    full = lambda r, c: pl.BlockSpec((r, c), lambda i: (0, 0))
    blk = lambda c: pl.BlockSpec((BR, c), lambda i: (i, 0))
    h2, h4, nc = H // 2, H // 4, 8
    return pl.pallas_call(
        _tc_cls_body,
        grid=(NB,),
        in_specs=[blk(H), full(H, h2), full(1, h2), full(1, h2), full(1, h2),
                  full(h2, h4), full(1, h4), full(1, h4), full(1, h4),
                  full(h4, nc), full(1, nc)],
        out_specs=blk(nc),
        out_shape=jax.ShapeDtypeStruct((N, nc), jnp.float32),
    )(h, Wc1, bc1.reshape(1, h2), g1.reshape(1, h2), be1.reshape(1, h2),
      Wc2, bc2.reshape(1, h4), g2.reshape(1, h4), be2.reshape(1, h4),
      Wc3, bc3.reshape(1, nc))


_MESH = plsc.VectorSubcoreMesh(core_axis_name="c", subcore_axis_name="s",
                               num_cores=NCORE, num_subcores=NSUB)
_GD = lax.GatherDimensionNumbers(offset_dims=(), collapsed_slice_dims=(0,),
                                 start_index_map=(0,))


@functools.partial(
    pl.kernel,
    out_type=(jax.ShapeDtypeStruct((NCORE, NSUB, RPT * H), jnp.float32),
              jax.ShapeDtypeStruct((NCORE, NSUB, DEN_W), jnp.float32)),
    mesh=_MESH,
    scratch_types=[
        pltpu.VMEM((RPT * H,), jnp.float32),
        pltpu.VMEM((DEN_W,), jnp.float32),
        pltpu.VMEM((DEN_W,), jnp.float32),
        pltpu.VMEM((32,), jnp.int32),
        pltpu.VMEM((GK,), jnp.int32),
        pltpu.VMEM((GK + 16,), jnp.int32),
        pltpu.VMEM((GK * 8 + 16,), jnp.float32),
        pltpu.VMEM((GK, H), jnp.float32),
        pltpu.SemaphoreType.DMA,
    ],
)
def _sc_edge(hw_hbm, ps_hbm, dtab_hbm, psrc_hbm, pdl_hbm, sbuf_hbm,
             acc0_hbm, den0_hbm, acc_out, den_out,
             acc_l, den_l, d_l, sbuf_v, idx_v, dl_v, ps_l, hwrow_v, sem):
    cid = lax.axis_index("c")
    sid = lax.axis_index("s")
    lane = lax.iota(jnp.int32, 16)

    pltpu.sync_copy(acc0_hbm.at[cid, sid], acc_l)
    pltpu.sync_copy(den0_hbm.at[cid, sid], den_l)
    pltpu.sync_copy(dtab_hbm.at[sid], d_l)
    pltpu.sync_copy(sbuf_hbm, sbuf_v)

    st = sbuf_v[pl.ds(sid, 16)][0]
    en = sbuf_v[pl.ds(sid + 1, 16)][0]
    h0 = (en - st) // 2
    my_lo = st + cid * h0
    my_hi = jnp.where(cid == 0, st + h0, en)
    base8 = (my_lo // 8) * 8

    @pl.loop(0, (my_hi - base8 + GK - 1) // GK)
    def _g(g):
        cb = base8 + g * GK
        pltpu.sync_copy(psrc_hbm.at[pl.ds(cb, GK)], idx_v)
        pltpu.sync_copy(pdl_hbm.at[pl.ds(cb, GK)], dl_v.at[pl.ds(0, GK)])
        pltpu.sync_copy(ps_hbm.at[pl.ds(cb * 8, GK * 8)],
                        ps_l.at[pl.ds(0, GK * 8)])
        pltpu.async_copy(hw_hbm.at[idx_v], hwrow_v, sem).wait()
        jlo = jnp.maximum(my_lo - cb, 0)
        jhi = jnp.minimum(my_hi - cb, GK)

        @pl.loop(jlo, jhi)
        def _e(j):
            dl = dl_v[pl.ds(j, 16)][0]
            sv = ps_l[pl.ds(j * 8, 16)]
            dv = d_l[pl.ds(dl * 8, 16)]
            e = sv + dv
            w16 = jnp.exp(jnp.maximum(e, 0.2 * e))
            w16 = jnp.where(lane < 8, w16, 0.0)
            db = den_l[pl.ds(dl * 8, 16)]
            den_l[pl.ds(dl * 8, 16)] = db + w16
            for c in range(8):
                cv = jnp.full((16,), c, jnp.int32)
                b = lax.gather(w16, cv[:, None], _GD, (1,),
                               mode=lax.GatherScatterMode.PROMISE_IN_BOUNDS)
                acc_l[pl.ds(dl * H + c * 16, 16)] = (
                    acc_l[pl.ds(dl * H + c * 16, 16)]
                    + hwrow_v[j, pl.ds(c * 16, 16)] * b)

    pltpu.sync_copy(acc_l, acc_out.at[cid, sid])
    pltpu.sync_copy(den_l, den_out.at[cid, sid])


def kernel(x, edge_index, W_in, b_in, g_in, be_in, W_l, a_src, a_dst, b_l,
           g_l, be_l, Wc1, bc1, g1, be1, Wc2, bc2, g2, be2, Wc3, bc3):
    src = edge_index[0]
    dst = edge_index[1]
    key = dst // RPT
    perm = jnp.argsort(key, stable=True)
    pad = jnp.zeros((GK + 8,), jnp.int32)
    psrc = jnp.concatenate([src[perm], pad])
    pdl = jnp.concatenate([dst[perm] - key[perm] * RPT, pad])
    skey = key[perm]
    starts = jnp.searchsorted(skey, jnp.arange(NSUB + 1, dtype=jnp.int32),
                              side="left").astype(jnp.int32)
    sbuf = jnp.concatenate([starts, jnp.zeros((32 - NSUB - 1,), jnp.int32)])
    eye = jnp.eye(HEADS, dtype=jnp.float32)
    R = jnp.repeat(eye, DH, axis=1)

    h = _tc_in(x, W_in, b_in, g_in, be_in)
    for i in range(3):
        As = (eye[:, None, :] * a_src[i][:, :, None]).reshape(H, HEADS)
        Ad = (eye[:, None, :] * a_dst[i][:, :, None]).reshape(H, HEADS)
        hw, s, ws, accs, dpad = _tc_pre(h, W_l[i], As, Ad, R)
        ps = jnp.concatenate([jnp.take(s, perm, axis=0).reshape(-1),
                              jnp.zeros(((GK + 8) * 8,), jnp.float32)])
        dtab = jnp.pad(dpad.reshape(NSUB, RPT * 8), ((0, 0), (0, 8)))
        acc_init = jnp.stack([accs.reshape(NSUB, RPT * H),
                              jnp.zeros((NSUB, RPT * H), jnp.float32)])
        den_init = jnp.stack([
            jnp.pad(ws.reshape(NSUB, RPT * 8), ((0, 0), (0, 8))),
            jnp.zeros((NSUB, DEN_W), jnp.float32)])
        accp, denp = _sc_edge(hw, ps, dtab, psrc, pdl, sbuf,
                              acc_init, den_init)
        accp = accp.reshape(NCORE, N, H)
        denp = denp[:, :, :RPT * 8].reshape(NCORE, N, HEADS)
        h = _tc_post(accp[0], accp[1], denp[0], denp[1], h,
                     b_l[i], g_l[i], be_l[i], R, add_resid=(i > 0))
    return _tc_cls(h, Wc1, bc1, g1, be1, Wc2, bc2, g2, be2, Wc3, bc3)
